# baseline (speedup 1.0000x reference)
import functools

import jax
import jax.numpy as jnp
from jax import lax
from jax.experimental import pallas as pl
from jax.experimental.pallas import tpu as pltpu
from jax.experimental.pallas import tpu_sc as plsc

_NC = 2
_NS = 16
_NW = _NC * _NS
_LANES = 16


def _sc_gather(table, idx, chunk):
    vocab = table.shape[0]
    n = idx.shape[0]
    n_per_w = n // _NW

    mesh = plsc.VectorSubcoreMesh(core_axis_name="c", subcore_axis_name="s")

    @functools.partial(
        pl.kernel,
        mesh=mesh,
        out_type=jax.ShapeDtypeStruct((n,), jnp.float32),
        scratch_types=[
            pltpu.VMEM((vocab,), jnp.float32),
            pltpu.VMEM((chunk,), jnp.int32),
            pltpu.VMEM((chunk,), jnp.float32),
        ],
        compiler_params=pltpu.CompilerParams(needs_layout_passes=False),
    )
    def gather_kernel(table_hbm, idx_hbm, out_hbm, table_v, idx_v, w_v):
        wid = lax.axis_index("s") * _NC + lax.axis_index("c")
        base = wid * n_per_w
        pltpu.sync_copy(table_hbm, table_v)

        def chunk_body(ci, carry):
            off = pl.multiple_of(base + ci * chunk, 8)
            pltpu.sync_copy(idx_hbm.at[pl.ds(off, chunk)], idx_v)

            def vec_body(i, c2):
                o16 = pl.multiple_of(i * _LANES, _LANES)
                vidx = idx_v[pl.ds(o16, _LANES)]
                w_v[pl.ds(o16, _LANES)] = plsc.load_gather(table_v, [vidx])
                return c2

            lax.fori_loop(0, chunk // _LANES, vec_body, 0)
            pltpu.sync_copy(w_v, out_hbm.at[pl.ds(off, chunk)])
            return carry

        lax.fori_loop(0, n_per_w // chunk, chunk_body, 0)

    return gather_kernel(table, idx)


def _rowsum_tc(emb2, rows=32768):
    n, d = emb2.shape

    def body(x_ref, o_ref):
        ones = jnp.ones((1, x_ref.shape[-1]), jnp.float32)
        s = jax.lax.dot_general(
            ones, x_ref[...],
            dimension_numbers=(((1,), (1,)), ((), ())),
            preferred_element_type=jnp.float32,
        )
        o_ref[...] = s[0, :]

    return pl.pallas_call(
        body,
        grid=(n // rows,),
        in_specs=[pl.BlockSpec((rows, d), lambda i: (i, 0))],
        out_specs=pl.BlockSpec((rows,), lambda i: (i,)),
        out_shape=jax.ShapeDtypeStruct((n,), jnp.float32),
    )(emb2)


def _mul_tc(w, s, rows=102400):
    n = w.shape[0]

    def body(w_ref, s_ref, o_ref):
        o_ref[...] = w_ref[...] * s_ref[...]

    return pl.pallas_call(
        body,
        grid=(n // rows,),
        in_specs=[
            pl.BlockSpec((rows,), lambda i: (i,)),
            pl.BlockSpec((rows,), lambda i: (i,)),
        ],
        out_specs=pl.BlockSpec((rows,), lambda i: (i,)),
        out_shape=jax.ShapeDtypeStruct((n,), jnp.float32),
    )(w, s)


def kernel(input_ids, embeddings, u_param):
    b, l = input_ids.shape
    d = embeddings.shape[-1]
    n = b * l
    idx = input_ids.reshape(n).astype(jnp.int32)
    emb2 = embeddings.reshape(n, d)
    w = _sc_gather(u_param.astype(jnp.float32), idx, chunk=6400)
    s = _rowsum_tc(emb2)
    out = _mul_tc(w, s)
    return out.reshape(b, l)


# speedup vs baseline: 1.0211x; 1.0211x over previous
import functools

import jax
import jax.numpy as jnp
from jax import lax
from jax.experimental import pallas as pl
from jax.experimental.pallas import tpu as pltpu
from jax.experimental.pallas import tpu_sc as plsc

_NC = 2
_NS = 16
_NW = _NC * _NS
_LANES = 16


def _sc_gather(table, idx, chunk):
    vocab = table.shape[0]
    n = idx.shape[0]
    n_per_w = n // _NW

    mesh = plsc.VectorSubcoreMesh(core_axis_name="c", subcore_axis_name="s")

    @functools.partial(
        pl.kernel,
        mesh=mesh,
        out_type=jax.ShapeDtypeStruct((n,), jnp.float32),
        scratch_types=[
            pltpu.VMEM((vocab,), jnp.float32),
            pltpu.VMEM((chunk,), jnp.int32),
            pltpu.VMEM((chunk,), jnp.float32),
        ],
        compiler_params=pltpu.CompilerParams(needs_layout_passes=False),
    )
    def gather_kernel(table_hbm, idx_hbm, out_hbm, table_v, idx_v, w_v):
        wid = lax.axis_index("s") * _NC + lax.axis_index("c")
        base = wid * n_per_w
        pltpu.sync_copy(table_hbm, table_v)

        def chunk_body(ci, carry):
            off = pl.multiple_of(base + ci * chunk, 8)
            pltpu.sync_copy(idx_hbm.at[pl.ds(off, chunk)], idx_v)

            def vec_body(i, c2):
                o16 = pl.multiple_of(i * _LANES, _LANES)
                vidx = idx_v[pl.ds(o16, _LANES)]
                w_v[pl.ds(o16, _LANES)] = plsc.load_gather(table_v, [vidx])
                return c2

            lax.fori_loop(0, chunk // _LANES, vec_body, 0)
            pltpu.sync_copy(w_v, out_hbm.at[pl.ds(off, chunk)])
            return carry

        lax.fori_loop(0, n_per_w // chunk, chunk_body, 0)

    return gather_kernel(table, idx)


def _rowsum_tc(emb2, rows=40960):
    n, d = emb2.shape

    def body(x_ref, o_ref):
        ones = jnp.ones((1, x_ref.shape[-1]), jnp.float32)
        s = jax.lax.dot_general(
            ones, x_ref[...],
            dimension_numbers=(((1,), (1,)), ((), ())),
            preferred_element_type=jnp.float32,
        )
        o_ref[...] = s[0, :]

    return pl.pallas_call(
        body,
        grid=(n // rows,),
        in_specs=[pl.BlockSpec((rows, d), lambda i: (i, 0))],
        out_specs=pl.BlockSpec((rows,), lambda i: (i,)),
        out_shape=jax.ShapeDtypeStruct((n,), jnp.float32),
    )(emb2)


def _mul_tc(w, s, rows=102400):
    n = w.shape[0]

    def body(w_ref, s_ref, o_ref):
        o_ref[...] = w_ref[...] * s_ref[...]

    return pl.pallas_call(
        body,
        grid=(n // rows,),
        in_specs=[
            pl.BlockSpec((rows,), lambda i: (i,)),
            pl.BlockSpec((rows,), lambda i: (i,)),
        ],
        out_specs=pl.BlockSpec((rows,), lambda i: (i,)),
        out_shape=jax.ShapeDtypeStruct((n,), jnp.float32),
    )(w, s)


def kernel(input_ids, embeddings, u_param):
    b, l = input_ids.shape
    d = embeddings.shape[-1]
    n = b * l
    idx = input_ids.reshape(n).astype(jnp.int32)
    emb2 = embeddings.reshape(n, d)
    w = _sc_gather(u_param.astype(jnp.float32), idx, chunk=6400)
    s = _rowsum_tc(emb2)
    out = _mul_tc(w, s)
    return out.reshape(b, l)


# speedup vs baseline: 1.0219x; 1.0008x over previous
import functools

import jax
import jax.numpy as jnp
from jax import lax
from jax.experimental import pallas as pl
from jax.experimental.pallas import tpu as pltpu
from jax.experimental.pallas import tpu_sc as plsc

_NC = 2
_NS = 16
_NW = _NC * _NS
_LANES = 16


def _sc_gather(table, idx, chunk):
    vocab = table.shape[0]
    n = idx.shape[0]
    n_per_w = n // _NW

    mesh = plsc.VectorSubcoreMesh(core_axis_name="c", subcore_axis_name="s")

    @functools.partial(
        pl.kernel,
        mesh=mesh,
        out_type=jax.ShapeDtypeStruct((n,), jnp.float32),
        scratch_types=[
            pltpu.VMEM((vocab,), jnp.float32),
            pltpu.VMEM((chunk,), jnp.int32),
            pltpu.VMEM((chunk,), jnp.float32),
        ],
        compiler_params=pltpu.CompilerParams(needs_layout_passes=False),
        cost_estimate=pl.CostEstimate(
            flops=0, bytes_accessed=32 * vocab * 4 + 2 * n * 4,
            transcendentals=0,
        ),
    )
    def gather_kernel(table_hbm, idx_hbm, out_hbm, table_v, idx_v, w_v):
        wid = lax.axis_index("s") * _NC + lax.axis_index("c")
        base = wid * n_per_w
        pltpu.sync_copy(table_hbm, table_v)

        def chunk_body(ci, carry):
            off = pl.multiple_of(base + ci * chunk, 8)
            pltpu.sync_copy(idx_hbm.at[pl.ds(off, chunk)], idx_v)

            def vec_body(i, c2):
                o16 = pl.multiple_of(i * _LANES, _LANES)
                vidx = idx_v[pl.ds(o16, _LANES)]
                w_v[pl.ds(o16, _LANES)] = plsc.load_gather(table_v, [vidx])
                return c2

            lax.fori_loop(0, chunk // _LANES, vec_body, 0)
            pltpu.sync_copy(w_v, out_hbm.at[pl.ds(off, chunk)])
            return carry

        lax.fori_loop(0, n_per_w // chunk, chunk_body, 0)

    return gather_kernel(table, idx)


def _rowsum_tc(emb2, rows=40960):
    n, d = emb2.shape

    def body(x_ref, o_ref):
        ones = jnp.ones((1, x_ref.shape[-1]), jnp.float32)
        s = jax.lax.dot_general(
            ones, x_ref[...],
            dimension_numbers=(((1,), (1,)), ((), ())),
            preferred_element_type=jnp.float32,
        )
        o_ref[...] = s[0, :]

    return pl.pallas_call(
        body,
        grid=(n // rows,),
        in_specs=[pl.BlockSpec((rows, d), lambda i: (i, 0))],
        out_specs=pl.BlockSpec((rows,), lambda i: (i,)),
        out_shape=jax.ShapeDtypeStruct((n,), jnp.float32),
        cost_estimate=pl.CostEstimate(
            flops=n * d, bytes_accessed=n * d * 4 + n * 4, transcendentals=0,
        ),
    )(emb2)


def _mul_tc(w, s, rows=102400):
    n = w.shape[0]

    def body(w_ref, s_ref, o_ref):
        o_ref[...] = w_ref[...] * s_ref[...]

    return pl.pallas_call(
        body,
        grid=(n // rows,),
        in_specs=[
            pl.BlockSpec((rows,), lambda i: (i,)),
            pl.BlockSpec((rows,), lambda i: (i,)),
        ],
        out_specs=pl.BlockSpec((rows,), lambda i: (i,)),
        out_shape=jax.ShapeDtypeStruct((n,), jnp.float32),
    )(w, s)


def kernel(input_ids, embeddings, u_param):
    b, l = input_ids.shape
    d = embeddings.shape[-1]
    n = b * l
    idx = input_ids.reshape(n).astype(jnp.int32)
    emb2 = embeddings.reshape(n, d)
    w = _sc_gather(u_param.astype(jnp.float32), idx, chunk=6400)
    s = _rowsum_tc(emb2)
    out = _mul_tc(w, s)
    return out.reshape(b, l)
